# Initial kernel scaffold; baseline (speedup 1.0000x reference)
import jax
import jax.numpy as jnp
from jax.experimental import pallas as pl


def kernel(patches, pos_table):
    raise NotImplementedError("write your pallas kernel here")


# speedup vs baseline: 1.0136x; 1.0136x over previous
import jax
import jax.numpy as jnp
from jax.experimental import pallas as pl


def _add_kernel(patches_ref, pos_ref, out_ref):
    out_ref[...] = patches_ref[...] + pos_ref[...]


def kernel(patches, pos_table):
    batch, num_patches, proj_dim = patches.shape
    grid = (batch,)
    return pl.pallas_call(
        _add_kernel,
        grid=grid,
        in_specs=[
            pl.BlockSpec((1, num_patches, proj_dim), lambda b: (b, 0, 0)),
            pl.BlockSpec((num_patches, proj_dim), lambda b: (0, 0)),
        ],
        out_specs=pl.BlockSpec((1, num_patches, proj_dim), lambda b: (b, 0, 0)),
        out_shape=jax.ShapeDtypeStruct(patches.shape, patches.dtype),
    )(patches, pos_table)


# speedup vs baseline: 1.0575x; 1.0433x over previous
import jax
import jax.numpy as jnp
from jax.experimental import pallas as pl


def _add_kernel(patches_ref, pos_ref, out_ref):
    out_ref[...] = patches_ref[...] + pos_ref[...]


def kernel(patches, pos_table):
    batch, num_patches, proj_dim = patches.shape
    bb = 4
    grid = (batch // bb,)
    return pl.pallas_call(
        _add_kernel,
        grid=grid,
        in_specs=[
            pl.BlockSpec((bb, num_patches, proj_dim), lambda b: (b, 0, 0)),
            pl.BlockSpec((num_patches, proj_dim), lambda b: (0, 0)),
        ],
        out_specs=pl.BlockSpec((bb, num_patches, proj_dim), lambda b: (b, 0, 0)),
        out_shape=jax.ShapeDtypeStruct(patches.shape, patches.dtype),
    )(patches, pos_table)
